# baseline (speedup 1.0000x reference)
import jax
import jax.numpy as jnp
from jax import lax
from jax.experimental import pallas as pl
from jax.experimental.pallas import tpu as pltpu

_LOG2 = 0.6931471805599453

_G = 1


def _ssp(x):
    return jnp.maximum(x, 0.0) + jnp.log(1.0 + jnp.exp(-jnp.abs(x))) - _LOG2


def _mlp(layers, x):
    for W, b in layers:
        x = _ssp(jnp.dot(x, W, preferred_element_type=jnp.float32) + b)
    return x


def _flatten_params(params):
    names = ['pre_e_che', 'pre_e_vdw', 'pre_v', 'pre_u',
             'phi_e_che', 'phi_e_vdw', 'phi_v_che', 'phi_v_vdw',
             'phi_u_che', 'phi_u_vdw']
    arrs, counts = [], {}
    for nm in names:
        layers = params[nm]
        counts[nm] = len(layers)
        for (W, b) in layers:
            arrs.append(W)
            arrs.append(b.reshape(1, -1))
    return names, arrs, counts


def kernel(nodes, state, che_edges, vdw_edges, che_index, vdw_index,
           che_edge_index, vdw_edge_index, node_index, num_atoms,
           che_num_pairs, vdw_num_pairs, params):
    N, H = nodes.shape
    B = state.shape[0]
    APG = N // B
    E_CHE = che_edges.shape[0]
    E_VDW = vdw_edges.shape[0]
    ECPG = E_CHE // B
    EVPG = E_VDW // B

    G = _G if B % _G == 0 else 1
    GA = G * APG
    GEC = G * ECPG
    GEV = G * EVPG

    names, warrs, counts = _flatten_params(params)

    nodes_b = nodes.reshape(B, APG, H)
    state_b = state.reshape(B, 1, H)
    che_b = che_edges.reshape(B, ECPG, H)
    vdw_b = vdw_edges.reshape(B, EVPG, H)
    chei_b = che_index.reshape(B, ECPG, 2)
    vdwi_b = vdw_index.reshape(B, EVPG, 2)

    def body(*refs):
        (nodes_ref, state_ref, che_ref, vdw_ref, chei_ref, vdwi_ref), rest = \
            refs[:6], refs[6:]
        nw = len(warrs)
        wrefs, outs = rest[:nw], rest[nw:]
        eche_o, evdw_o, v_o, u_o = outs

        layers = {}
        k = 0
        for nm in names:
            ls = []
            for _ in range(counts[nm]):
                ls.append((wrefs[k][...], wrefs[k + 1][...]))
                k += 2
            layers[nm] = ls

        pid = pl.program_id(0)
        base = pid * GA

        v_in = nodes_ref[...].reshape(GA, H)
        u_in = state_ref[...].reshape(G, H)
        eche_in = che_ref[...].reshape(GEC, H)
        evdw_in = vdw_ref[...].reshape(GEV, H)

        v = _mlp(layers['pre_v'], v_in)
        u = _mlp(layers['pre_u'], u_in)
        e_che = _mlp(layers['pre_e_che'], eche_in)
        e_vdw = _mlp(layers['pre_e_vdw'], evdw_in)

        u_pernode = jnp.broadcast_to(u[:, None, :], (G, APG, H)).reshape(GA, H)

        def half(e, idx_ref, EPG, GE, phi_e, phi_v, phi_u):
            idxl = idx_ref[...] - base
            iota3 = lax.broadcasted_iota(jnp.int32, (G, EPG, GA), 2)
            ohg0 = (idxl[:, :, 0:1] == iota3).astype(jnp.float32).reshape(GE, GA)
            ohg1 = (idxl[:, :, 1:2] == iota3).astype(jnp.float32).reshape(GE, GA)
            center = jnp.dot(ohg0, v, preferred_element_type=jnp.float32)
            nbr = jnp.dot(ohg1, v, preferred_element_type=jnp.float32)
            u_e = jnp.broadcast_to(u[:, None, :], (G, EPG, H)).reshape(GE, H)
            e_p = _mlp(phi_e, jnp.concatenate([center, e, nbr, u_e], axis=-1))
            row0 = idxl[:, :, 0].reshape(1, GE)
            ohs0 = (lax.broadcasted_iota(jnp.int32, (GA, GE), 0) == row0
                    ).astype(jnp.float32)
            e_v = jnp.dot(ohs0, e_p, preferred_element_type=jnp.float32)
            v_p = _mlp(phi_v, jnp.concatenate([e_v, v, u_pernode], axis=-1))
            e_u = jnp.sum(e_p.reshape(G, EPG, H), axis=1)
            v_u = jnp.sum(v_p.reshape(G, APG, H), axis=1)
            u_p = _mlp(phi_u, jnp.concatenate([e_u, v_u, u], axis=-1))
            return e_p, v_p, u_p

        che_e_p, che_v_p, che_u_p = half(
            e_che, chei_ref, ECPG, GEC,
            layers['phi_e_che'], layers['phi_v_che'], layers['phi_u_che'])
        vdw_e_p, vdw_v_p, vdw_u_p = half(
            e_vdw, vdwi_ref, EVPG, GEV,
            layers['phi_e_vdw'], layers['phi_v_vdw'], layers['phi_u_vdw'])

        eche_o[...] = (eche_in + che_e_p).reshape(G, ECPG, H)
        evdw_o[...] = (evdw_in + vdw_e_p).reshape(G, EVPG, H)
        v_o[...] = (v_in + che_v_p + vdw_v_p).reshape(G, APG, H)
        u_o[...] = (u_in + che_u_p + vdw_u_p).reshape(G, 1, H)

    grid = (B // G,)

    def blk(shape):
        nd = len(shape)
        return pl.BlockSpec((G,) + shape[1:], lambda i: (i,) + (0,) * (nd - 1))

    def full(shape):
        nd = len(shape)
        return pl.BlockSpec(shape, lambda i: (0,) * nd)

    in_specs = [
        blk(nodes_b.shape), blk(state_b.shape), blk(che_b.shape),
        blk(vdw_b.shape), blk(chei_b.shape), blk(vdwi_b.shape),
    ] + [full(w.shape) for w in warrs]

    out_shape = (
        jax.ShapeDtypeStruct((B, ECPG, H), jnp.float32),
        jax.ShapeDtypeStruct((B, EVPG, H), jnp.float32),
        jax.ShapeDtypeStruct((B, APG, H), jnp.float32),
        jax.ShapeDtypeStruct((B, 1, H), jnp.float32),
    )
    out_specs = (
        blk((B, ECPG, H)), blk((B, EVPG, H)), blk((B, APG, H)), blk((B, 1, H)),
    )

    eche_o, evdw_o, v_o, u_o = pl.pallas_call(
        body,
        grid=grid,
        in_specs=in_specs,
        out_specs=out_specs,
        out_shape=out_shape,
        compiler_params=pltpu.CompilerParams(
            dimension_semantics=("arbitrary",),
        ),
    )(nodes_b, state_b, che_b, vdw_b, chei_b, vdwi_b, *warrs)

    return (eche_o.reshape(E_CHE, H), evdw_o.reshape(E_VDW, H),
            v_o.reshape(N, H), u_o.reshape(B, H))


# speedup vs baseline: 1.0718x; 1.0718x over previous
import jax
import jax.numpy as jnp
from jax import lax
from jax.experimental import pallas as pl
from jax.experimental.pallas import tpu as pltpu

_LOG2 = 0.6931471805599453

_G = 4


def _ssp(x):
    return jnp.maximum(x, 0.0) + jnp.log(1.0 + jnp.exp(-jnp.abs(x))) - _LOG2


def _mlp(layers, x):
    for W, b in layers:
        x = _ssp(jnp.dot(x, W, preferred_element_type=jnp.float32) + b)
    return x


def _flatten_params(params):
    names = ['pre_e_che', 'pre_e_vdw', 'pre_v', 'pre_u',
             'phi_e_che', 'phi_e_vdw', 'phi_v_che', 'phi_v_vdw',
             'phi_u_che', 'phi_u_vdw']
    arrs, counts = [], {}
    for nm in names:
        layers = params[nm]
        counts[nm] = len(layers)
        for (W, b) in layers:
            arrs.append(W)
            arrs.append(b.reshape(1, -1))
    return names, arrs, counts


def kernel(nodes, state, che_edges, vdw_edges, che_index, vdw_index,
           che_edge_index, vdw_edge_index, node_index, num_atoms,
           che_num_pairs, vdw_num_pairs, params):
    N, H = nodes.shape
    B = state.shape[0]
    APG = N // B
    E_CHE = che_edges.shape[0]
    E_VDW = vdw_edges.shape[0]
    ECPG = E_CHE // B
    EVPG = E_VDW // B

    G = _G if B % _G == 0 else 1
    GA = G * APG
    GEC = G * ECPG
    GEV = G * EVPG

    names, warrs, counts = _flatten_params(params)

    nodes_b = nodes.reshape(B, APG, H)
    state_b = state.reshape(B, 1, H)
    che_b = che_edges.reshape(B, ECPG, H)
    vdw_b = vdw_edges.reshape(B, EVPG, H)
    chei_b = che_index.reshape(B, ECPG, 2)
    vdwi_b = vdw_index.reshape(B, EVPG, 2)

    def body(*refs):
        (nodes_ref, state_ref, che_ref, vdw_ref, chei_ref, vdwi_ref), rest = \
            refs[:6], refs[6:]
        nw = len(warrs)
        wrefs, outs = rest[:nw], rest[nw:]
        eche_o, evdw_o, v_o, u_o = outs

        layers = {}
        k = 0
        for nm in names:
            ls = []
            for _ in range(counts[nm]):
                ls.append((wrefs[k][...], wrefs[k + 1][...]))
                k += 2
            layers[nm] = ls

        pid = pl.program_id(0)
        base = pid * GA

        v_in = nodes_ref[...].reshape(GA, H)
        u_in = state_ref[...].reshape(G, H)
        eche_in = che_ref[...].reshape(GEC, H)
        evdw_in = vdw_ref[...].reshape(GEV, H)

        v = _mlp(layers['pre_v'], v_in)
        u = _mlp(layers['pre_u'], u_in)
        e_che = _mlp(layers['pre_e_che'], eche_in)
        e_vdw = _mlp(layers['pre_e_vdw'], evdw_in)

        u_pernode = jnp.broadcast_to(u[:, None, :], (G, APG, H)).reshape(GA, H)

        def half(e, idx_ref, EPG, GE, phi_e, phi_v, phi_u):
            idxl = idx_ref[...] - base
            iota3 = lax.broadcasted_iota(jnp.int32, (G, EPG, GA), 2)
            ohg0 = (idxl[:, :, 0:1] == iota3).astype(jnp.float32).reshape(GE, GA)
            ohg1 = (idxl[:, :, 1:2] == iota3).astype(jnp.float32).reshape(GE, GA)
            center = jnp.dot(ohg0, v, preferred_element_type=jnp.float32)
            nbr = jnp.dot(ohg1, v, preferred_element_type=jnp.float32)
            u_e = jnp.broadcast_to(u[:, None, :], (G, EPG, H)).reshape(GE, H)
            e_p = _mlp(phi_e, jnp.concatenate([center, e, nbr, u_e], axis=-1))
            row0 = idxl[:, :, 0].reshape(1, GE)
            ohs0 = (lax.broadcasted_iota(jnp.int32, (GA, GE), 0) == row0
                    ).astype(jnp.float32)
            e_v = jnp.dot(ohs0, e_p, preferred_element_type=jnp.float32)
            v_p = _mlp(phi_v, jnp.concatenate([e_v, v, u_pernode], axis=-1))
            e_u = jnp.sum(e_p.reshape(G, EPG, H), axis=1)
            v_u = jnp.sum(v_p.reshape(G, APG, H), axis=1)
            u_p = _mlp(phi_u, jnp.concatenate([e_u, v_u, u], axis=-1))
            return e_p, v_p, u_p

        che_e_p, che_v_p, che_u_p = half(
            e_che, chei_ref, ECPG, GEC,
            layers['phi_e_che'], layers['phi_v_che'], layers['phi_u_che'])
        vdw_e_p, vdw_v_p, vdw_u_p = half(
            e_vdw, vdwi_ref, EVPG, GEV,
            layers['phi_e_vdw'], layers['phi_v_vdw'], layers['phi_u_vdw'])

        eche_o[...] = (eche_in + che_e_p).reshape(G, ECPG, H)
        evdw_o[...] = (evdw_in + vdw_e_p).reshape(G, EVPG, H)
        v_o[...] = (v_in + che_v_p + vdw_v_p).reshape(G, APG, H)
        u_o[...] = (u_in + che_u_p + vdw_u_p).reshape(G, 1, H)

    grid = (B // G,)

    def blk(shape):
        nd = len(shape)
        return pl.BlockSpec((G,) + shape[1:], lambda i: (i,) + (0,) * (nd - 1))

    def full(shape):
        nd = len(shape)
        return pl.BlockSpec(shape, lambda i: (0,) * nd)

    in_specs = [
        blk(nodes_b.shape), blk(state_b.shape), blk(che_b.shape),
        blk(vdw_b.shape), blk(chei_b.shape), blk(vdwi_b.shape),
    ] + [full(w.shape) for w in warrs]

    out_shape = (
        jax.ShapeDtypeStruct((B, ECPG, H), jnp.float32),
        jax.ShapeDtypeStruct((B, EVPG, H), jnp.float32),
        jax.ShapeDtypeStruct((B, APG, H), jnp.float32),
        jax.ShapeDtypeStruct((B, 1, H), jnp.float32),
    )
    out_specs = (
        blk((B, ECPG, H)), blk((B, EVPG, H)), blk((B, APG, H)), blk((B, 1, H)),
    )

    eche_o, evdw_o, v_o, u_o = pl.pallas_call(
        body,
        grid=grid,
        in_specs=in_specs,
        out_specs=out_specs,
        out_shape=out_shape,
        compiler_params=pltpu.CompilerParams(
            dimension_semantics=("arbitrary",),
        ),
    )(nodes_b, state_b, che_b, vdw_b, chei_b, vdwi_b, *warrs)

    return (eche_o.reshape(E_CHE, H), evdw_o.reshape(E_VDW, H),
            v_o.reshape(N, H), u_o.reshape(B, H))


# speedup vs baseline: 1.2539x; 1.1698x over previous
import jax
import jax.numpy as jnp
from jax import lax
from jax.experimental import pallas as pl
from jax.experimental.pallas import tpu as pltpu

_LOG2 = 0.6931471805599453

_G = 2


def _ssp(x):
    return jnp.maximum(x, 0.0) + jnp.log(1.0 + jnp.exp(-jnp.abs(x))) - _LOG2


def _ssp_fast(x):
    return jnp.log(1.0 + jnp.exp(x)) - _LOG2


def _mlp(layers, x, act=_ssp):
    for W, b in layers:
        x = act(jnp.dot(x, W, preferred_element_type=jnp.float32) + b)
    return x


def _flatten_params(params):
    names = ['pre_e_che', 'pre_e_vdw', 'pre_v', 'pre_u',
             'phi_e_che', 'phi_e_vdw', 'phi_v_che', 'phi_v_vdw',
             'phi_u_che', 'phi_u_vdw']
    arrs, counts = [], {}
    for nm in names:
        layers = params[nm]
        counts[nm] = len(layers)
        for (W, b) in layers:
            arrs.append(W)
            arrs.append(b.reshape(1, -1))
    return names, arrs, counts


def kernel(nodes, state, che_edges, vdw_edges, che_index, vdw_index,
           che_edge_index, vdw_edge_index, node_index, num_atoms,
           che_num_pairs, vdw_num_pairs, params):
    N, H = nodes.shape
    B = state.shape[0]
    APG = N // B
    E_CHE = che_edges.shape[0]
    E_VDW = vdw_edges.shape[0]
    ECPG = E_CHE // B
    EVPG = E_VDW // B

    G = _G if B % _G == 0 else 1
    GA = G * APG
    GEC = G * ECPG
    GEV = G * EVPG

    names, warrs, counts = _flatten_params(params)

    nodes_b = nodes.reshape(B, APG, H)
    state_b = state.reshape(B, 1, H)
    che_b = che_edges.reshape(B, ECPG, H)
    vdw_b = vdw_edges.reshape(B, EVPG, H)
    chei_b = che_index.reshape(B, ECPG, 2)
    vdwi_b = vdw_index.reshape(B, EVPG, 2)

    def body(*refs):
        (nodes_ref, state_ref, che_ref, vdw_ref, chei_ref, vdwi_ref), rest = \
            refs[:6], refs[6:]
        nw = len(warrs)
        wrefs, outs = rest[:nw], rest[nw:]
        eche_o, evdw_o, v_o, u_o = outs

        layers = {}
        k = 0
        for nm in names:
            ls = []
            for _ in range(counts[nm]):
                ls.append((wrefs[k][...], wrefs[k + 1][...]))
                k += 2
            layers[nm] = ls

        pid = pl.program_id(0)
        base = pid * GA

        v_in = nodes_ref[...].reshape(GA, H)
        u_in = state_ref[...].reshape(G, H)
        eche_in = che_ref[...].reshape(GEC, H)
        evdw_in = vdw_ref[...].reshape(GEV, H)

        v = _mlp(layers['pre_v'], v_in, _ssp_fast)
        u = _mlp(layers['pre_u'], u_in, _ssp_fast)
        e_che = _mlp(layers['pre_e_che'], eche_in, _ssp_fast)
        e_vdw = _mlp(layers['pre_e_vdw'], evdw_in, _ssp_fast)

        u_pernode = jnp.broadcast_to(u[:, None, :], (G, APG, H)).reshape(GA, H)

        def half(e, idx_ref, EPG, GE, phi_e, phi_v, phi_u):
            idxl = idx_ref[...] - base
            iota3 = lax.broadcasted_iota(jnp.int32, (G, EPG, GA), 2)
            ohg0 = (idxl[:, :, 0:1] == iota3).astype(jnp.float32).reshape(GE, GA)
            ohg1 = (idxl[:, :, 1:2] == iota3).astype(jnp.float32).reshape(GE, GA)
            center = jnp.dot(ohg0, v, preferred_element_type=jnp.float32)
            nbr = jnp.dot(ohg1, v, preferred_element_type=jnp.float32)
            u_e = jnp.broadcast_to(u[:, None, :], (G, EPG, H)).reshape(GE, H)
            e_p = _mlp(phi_e, jnp.concatenate([center, e, nbr, u_e], axis=-1), _ssp_fast)
            row0 = idxl[:, :, 0].reshape(1, GE)
            ohs0 = (lax.broadcasted_iota(jnp.int32, (GA, GE), 0) == row0
                    ).astype(jnp.float32)
            e_v = jnp.dot(ohs0, e_p, preferred_element_type=jnp.float32)
            v_p = _mlp(phi_v, jnp.concatenate([e_v, v, u_pernode], axis=-1), _ssp_fast)
            e_u = jnp.sum(e_p.reshape(G, EPG, H), axis=1)
            v_u = jnp.sum(v_p.reshape(G, APG, H), axis=1)
            u_p = _mlp(phi_u, jnp.concatenate([e_u, v_u, u], axis=-1))
            return e_p, v_p, u_p

        che_e_p, che_v_p, che_u_p = half(
            e_che, chei_ref, ECPG, GEC,
            layers['phi_e_che'], layers['phi_v_che'], layers['phi_u_che'])
        vdw_e_p, vdw_v_p, vdw_u_p = half(
            e_vdw, vdwi_ref, EVPG, GEV,
            layers['phi_e_vdw'], layers['phi_v_vdw'], layers['phi_u_vdw'])

        eche_o[...] = (eche_in + che_e_p).reshape(G, ECPG, H)
        evdw_o[...] = (evdw_in + vdw_e_p).reshape(G, EVPG, H)
        v_o[...] = (v_in + che_v_p + vdw_v_p).reshape(G, APG, H)
        u_o[...] = (u_in + che_u_p + vdw_u_p).reshape(G, 1, H)

    grid = (B // G,)

    def blk(shape):
        nd = len(shape)
        return pl.BlockSpec((G,) + shape[1:], lambda i: (i,) + (0,) * (nd - 1))

    def full(shape):
        nd = len(shape)
        return pl.BlockSpec(shape, lambda i: (0,) * nd)

    in_specs = [
        blk(nodes_b.shape), blk(state_b.shape), blk(che_b.shape),
        blk(vdw_b.shape), blk(chei_b.shape), blk(vdwi_b.shape),
    ] + [full(w.shape) for w in warrs]

    out_shape = (
        jax.ShapeDtypeStruct((B, ECPG, H), jnp.float32),
        jax.ShapeDtypeStruct((B, EVPG, H), jnp.float32),
        jax.ShapeDtypeStruct((B, APG, H), jnp.float32),
        jax.ShapeDtypeStruct((B, 1, H), jnp.float32),
    )
    out_specs = (
        blk((B, ECPG, H)), blk((B, EVPG, H)), blk((B, APG, H)), blk((B, 1, H)),
    )

    eche_o, evdw_o, v_o, u_o = pl.pallas_call(
        body,
        grid=grid,
        in_specs=in_specs,
        out_specs=out_specs,
        out_shape=out_shape,
        compiler_params=pltpu.CompilerParams(
            dimension_semantics=("arbitrary",),
        ),
    )(nodes_b, state_b, che_b, vdw_b, chei_b, vdwi_b, *warrs)

    return (eche_o.reshape(E_CHE, H), evdw_o.reshape(E_VDW, H),
            v_o.reshape(N, H), u_o.reshape(B, H))


# speedup vs baseline: 1.3373x; 1.0665x over previous
import jax
import jax.numpy as jnp
from jax import lax
from jax.experimental import pallas as pl
from jax.experimental.pallas import tpu as pltpu

_LOG2 = 0.6931471805599453

_G = 2


def _ssp(x):
    return jnp.maximum(x, 0.0) + jnp.log(1.0 + jnp.exp(-jnp.abs(x))) - _LOG2


def _ssp_fast(x):
    return jnp.log(1.0 + jnp.exp(x)) - _LOG2


def _mlp(layers, x, act=_ssp):
    for W, b in layers:
        x = act(jnp.dot(x, W, preferred_element_type=jnp.float32) + b)
    return x


def _flatten_params(params):
    names = ['pre_e_che', 'pre_e_vdw', 'pre_v', 'pre_u',
             'phi_e_che', 'phi_e_vdw', 'phi_v_che', 'phi_v_vdw',
             'phi_u_che', 'phi_u_vdw']
    arrs, counts = [], {}
    for nm in names:
        layers = params[nm]
        counts[nm] = len(layers)
        for (W, b) in layers:
            arrs.append(W)
            arrs.append(b.reshape(1, -1))
    return names, arrs, counts


def kernel(nodes, state, che_edges, vdw_edges, che_index, vdw_index,
           che_edge_index, vdw_edge_index, node_index, num_atoms,
           che_num_pairs, vdw_num_pairs, params):
    N, H = nodes.shape
    B = state.shape[0]
    APG = N // B
    E_CHE = che_edges.shape[0]
    E_VDW = vdw_edges.shape[0]
    ECPG = E_CHE // B
    EVPG = E_VDW // B

    G = _G if B % _G == 0 else 1
    GA = G * APG
    GEC = G * ECPG
    GEV = G * EVPG

    names, warrs, counts = _flatten_params(params)

    state_b = state.reshape(B, 1, H)

    def body(*refs):
        (nodes_ref, state_ref, che_ref, vdw_ref, chei_ref, vdwi_ref), rest = \
            refs[:6], refs[6:]
        nw = len(warrs)
        wrefs, outs = rest[:nw], rest[nw:]
        eche_o, evdw_o, v_o, u_o = outs

        layers = {}
        k = 0
        for nm in names:
            ls = []
            for _ in range(counts[nm]):
                ls.append((wrefs[k][...], wrefs[k + 1][...]))
                k += 2
            layers[nm] = ls

        pid = pl.program_id(0)
        base = pid * GA

        v_in = nodes_ref[...]
        u_in = state_ref[...].reshape(G, H)
        eche_in = che_ref[...]
        evdw_in = vdw_ref[...]

        v = _mlp(layers['pre_v'], v_in, _ssp_fast)
        u = _mlp(layers['pre_u'], u_in, _ssp_fast)
        e_che = _mlp(layers['pre_e_che'], eche_in, _ssp_fast)
        e_vdw = _mlp(layers['pre_e_vdw'], evdw_in, _ssp_fast)

        u_pernode = jnp.broadcast_to(u[:, None, :], (G, APG, H)).reshape(GA, H)

        def half(e, idx_ref, EPG, GE, phi_e, phi_v, phi_u):
            idxl = idx_ref[...] - base
            iota2 = lax.broadcasted_iota(jnp.int32, (GE, GA), 1)
            ohg0 = (idxl[:, 0:1] == iota2).astype(jnp.float32)
            ohg1 = (idxl[:, 1:2] == iota2).astype(jnp.float32)
            center = jnp.dot(ohg0, v, preferred_element_type=jnp.float32)
            nbr = jnp.dot(ohg1, v, preferred_element_type=jnp.float32)
            u_e = jnp.broadcast_to(u[:, None, :], (G, EPG, H)).reshape(GE, H)
            e_p = _mlp(phi_e, jnp.concatenate([center, e, nbr, u_e], axis=-1), _ssp_fast)
            row0 = idxl[:, 0].reshape(1, GE)
            ohs0 = (lax.broadcasted_iota(jnp.int32, (GA, GE), 0) == row0
                    ).astype(jnp.float32)
            e_v = jnp.dot(ohs0, e_p, preferred_element_type=jnp.float32)
            v_p = _mlp(phi_v, jnp.concatenate([e_v, v, u_pernode], axis=-1), _ssp_fast)
            e_u = jnp.sum(e_p.reshape(G, EPG, H), axis=1)
            v_u = jnp.sum(v_p.reshape(G, APG, H), axis=1)
            u_p = _mlp(phi_u, jnp.concatenate([e_u, v_u, u], axis=-1))
            return e_p, v_p, u_p

        che_e_p, che_v_p, che_u_p = half(
            e_che, chei_ref, ECPG, GEC,
            layers['phi_e_che'], layers['phi_v_che'], layers['phi_u_che'])
        vdw_e_p, vdw_v_p, vdw_u_p = half(
            e_vdw, vdwi_ref, EVPG, GEV,
            layers['phi_e_vdw'], layers['phi_v_vdw'], layers['phi_u_vdw'])

        eche_o[...] = eche_in + che_e_p
        evdw_o[...] = evdw_in + vdw_e_p
        v_o[...] = v_in + che_v_p + vdw_v_p
        u_o[...] = (u_in + che_u_p + vdw_u_p).reshape(G, 1, H)

    grid = (B // G,)

    def blk(shape):
        nd = len(shape)
        return pl.BlockSpec((G,) + shape[1:], lambda i: (i,) + (0,) * (nd - 1))

    def full(shape):
        nd = len(shape)
        return pl.BlockSpec(shape, lambda i: (0,) * nd)

    def row_blk(rows, cols):
        return pl.BlockSpec((rows, cols), lambda i: (i, 0))

    in_specs = [
        row_blk(GA, H), blk(state_b.shape), row_blk(GEC, H),
        row_blk(GEV, H), row_blk(GEC, 2), row_blk(GEV, 2),
    ] + [full(w.shape) for w in warrs]

    out_shape = (
        jax.ShapeDtypeStruct((E_CHE, H), jnp.float32),
        jax.ShapeDtypeStruct((E_VDW, H), jnp.float32),
        jax.ShapeDtypeStruct((N, H), jnp.float32),
        jax.ShapeDtypeStruct((B, 1, H), jnp.float32),
    )
    out_specs = (
        row_blk(GEC, H), row_blk(GEV, H), row_blk(GA, H), blk((B, 1, H)),
    )

    eche_o, evdw_o, v_o, u_o = pl.pallas_call(
        body,
        grid=grid,
        in_specs=in_specs,
        out_specs=out_specs,
        out_shape=out_shape,
        compiler_params=pltpu.CompilerParams(
            dimension_semantics=("arbitrary",),
        ),
    )(nodes, state_b, che_edges, vdw_edges, che_index, vdw_index, *warrs)

    return (eche_o, evdw_o, v_o, u_o.reshape(B, H))


# speedup vs baseline: 1.4584x; 1.0905x over previous
import jax
import jax.numpy as jnp
from jax import lax
from jax.experimental import pallas as pl
from jax.experimental.pallas import tpu as pltpu

_LOG2 = 0.6931471805599453

_G = 2


def _ssp(x):
    return jnp.maximum(x, 0.0) + jnp.log(1.0 + jnp.exp(-jnp.abs(x))) - _LOG2


def _ssp_fast(x):
    return jnp.log(1.0 + jnp.exp(x)) - _LOG2


def _mlp(layers, x, act=_ssp):
    for W, b in layers:
        x = act(jnp.dot(x, W, preferred_element_type=jnp.float32) + b)
    return x


def _flatten_params(params):
    names = ['pre_e_che', 'pre_e_vdw', 'pre_v', 'pre_u',
             'phi_e_che', 'phi_e_vdw', 'phi_v_che', 'phi_v_vdw',
             'phi_u_che', 'phi_u_vdw']
    arrs, counts = [], {}
    for nm in names:
        layers = params[nm]
        counts[nm] = len(layers)
        for (W, b) in layers:
            arrs.append(W)
            arrs.append(b.reshape(1, -1))
    return names, arrs, counts


def kernel(nodes, state, che_edges, vdw_edges, che_index, vdw_index,
           che_edge_index, vdw_edge_index, node_index, num_atoms,
           che_num_pairs, vdw_num_pairs, params):
    N, H = nodes.shape
    B = state.shape[0]
    APG = N // B
    E_CHE = che_edges.shape[0]
    E_VDW = vdw_edges.shape[0]
    ECPG = E_CHE // B
    EVPG = E_VDW // B

    G = _G if B % _G == 0 else 1
    GA = G * APG
    GEC = G * ECPG
    GEV = G * EVPG

    names, warrs, counts = _flatten_params(params)

    state_b = state.reshape(B, 1, H)
    chei_t = che_index.T.reshape(2, B // G, GEC).transpose(1, 0, 2)
    vdwi_t = vdw_index.T.reshape(2, B // G, GEV).transpose(1, 0, 2)
    cc = GEC // 8
    cv = GEV // 8
    che_i0 = che_index[:, 0].reshape(E_CHE // cc, cc)
    che_i1 = che_index[:, 1].reshape(E_CHE // cc, cc)
    vdw_i0 = vdw_index[:, 0].reshape(E_VDW // cv, cv)
    vdw_i1 = vdw_index[:, 1].reshape(E_VDW // cv, cv)

    def body(*refs):
        (nodes_ref, state_ref, che_ref, vdw_ref, chei_ref, vdwi_ref), rest = \
            refs[:6], refs[6:]
        nw = len(warrs)
        wrefs, outs = rest[:nw], rest[nw:]
        eche_o, evdw_o, v_o, u_o = outs

        layers = {}
        k = 0
        for nm in names:
            ls = []
            for _ in range(counts[nm]):
                ls.append((wrefs[k][...], wrefs[k + 1][...]))
                k += 2
            layers[nm] = ls

        pid = pl.program_id(0)
        base = pid * GA

        v_in = nodes_ref[...]
        u_in = state_ref[...].reshape(G, H)
        eche_in = che_ref[...]
        evdw_in = vdw_ref[...]

        v = _mlp(layers['pre_v'], v_in, _ssp_fast)
        u = _mlp(layers['pre_u'], u_in, _ssp_fast)
        e_che = _mlp(layers['pre_e_che'], eche_in, _ssp_fast)
        e_vdw = _mlp(layers['pre_e_vdw'], evdw_in, _ssp_fast)

        u_pernode = jnp.broadcast_to(u[:, None, :], (G, APG, H)).reshape(GA, H)

        def half(e, idx_ref, EPG, GE, phi_e, phi_v, phi_u):
            idxt = idx_ref[...].reshape(2, GE) - base
            iota_t = lax.broadcasted_iota(jnp.int32, (GA, GE), 0)
            ohg0t = (iota_t == idxt[0:1, :]).astype(jnp.float32)
            ohg1t = (iota_t == idxt[1:2, :]).astype(jnp.float32)
            tdot = (((0,), (0,)), ((), ()))
            center = lax.dot_general(ohg0t, v, tdot,
                                     preferred_element_type=jnp.float32)
            nbr = lax.dot_general(ohg1t, v, tdot,
                                  preferred_element_type=jnp.float32)
            u_e = jnp.broadcast_to(u[:, None, :], (G, EPG, H)).reshape(GE, H)
            e_p = _mlp(phi_e, jnp.concatenate([center, e, nbr, u_e], axis=-1), _ssp_fast)
            e_v = jnp.dot(ohg0t, e_p, preferred_element_type=jnp.float32)
            v_p = _mlp(phi_v, jnp.concatenate([e_v, v, u_pernode], axis=-1), _ssp_fast)
            e_u = jnp.sum(e_p.reshape(G, EPG, H), axis=1)
            v_u = jnp.sum(v_p.reshape(G, APG, H), axis=1)
            u_p = _mlp(phi_u, jnp.concatenate([e_u, v_u, u], axis=-1))
            return e_p, v_p, u_p

        che_e_p, che_v_p, che_u_p = half(
            e_che, chei_ref, ECPG, GEC,
            layers['phi_e_che'], layers['phi_v_che'], layers['phi_u_che'])
        vdw_e_p, vdw_v_p, vdw_u_p = half(
            e_vdw, vdwi_ref, EVPG, GEV,
            layers['phi_e_vdw'], layers['phi_v_vdw'], layers['phi_u_vdw'])

        eche_o[...] = eche_in + che_e_p
        evdw_o[...] = evdw_in + vdw_e_p
        v_o[...] = v_in + che_v_p + vdw_v_p
        u_o[...] = (u_in + che_u_p + vdw_u_p).reshape(G, 1, H)

    grid = (B // G,)

    def blk(shape):
        nd = len(shape)
        return pl.BlockSpec((G,) + shape[1:], lambda i: (i,) + (0,) * (nd - 1))

    def full(shape):
        nd = len(shape)
        return pl.BlockSpec(shape, lambda i: (0,) * nd)

    def row_blk(rows, cols):
        return pl.BlockSpec((rows, cols), lambda i: (i, 0))

    in_specs = [
        row_blk(GA, H), blk(state_b.shape), row_blk(GEC, H),
        row_blk(GEV, H),
        pl.BlockSpec((1, 2, GEC), lambda i: (i, 0, 0)),
        pl.BlockSpec((1, 2, GEV), lambda i: (i, 0, 0)),
    ] + [full(w.shape) for w in warrs]

    out_shape = (
        jax.ShapeDtypeStruct((E_CHE, H), jnp.float32),
        jax.ShapeDtypeStruct((E_VDW, H), jnp.float32),
        jax.ShapeDtypeStruct((N, H), jnp.float32),
        jax.ShapeDtypeStruct((B, 1, H), jnp.float32),
    )
    out_specs = (
        row_blk(GEC, H), row_blk(GEV, H), row_blk(GA, H), blk((B, 1, H)),
    )

    eche_o, evdw_o, v_o, u_o = pl.pallas_call(
        body,
        grid=grid,
        in_specs=in_specs,
        out_specs=out_specs,
        out_shape=out_shape,
        compiler_params=pltpu.CompilerParams(
            dimension_semantics=("arbitrary",),
        ),
    )(nodes, state_b, che_edges, vdw_edges, chei_t, vdwi_t, *warrs)

    return (eche_o, evdw_o, v_o, u_o.reshape(B, H))


# speedup vs baseline: 1.4601x; 1.0012x over previous
import jax
import jax.numpy as jnp
from jax import lax
from jax.experimental import pallas as pl
from jax.experimental.pallas import tpu as pltpu

_LOG2 = 0.6931471805599453

_G = 2


def _ssp(x):
    return jnp.maximum(x, 0.0) + jnp.log(1.0 + jnp.exp(-jnp.abs(x))) - _LOG2


def _ssp_fast(x):
    return jnp.log(1.0 + jnp.exp(x)) - _LOG2


def _mlp(layers, x, act=_ssp):
    for W, b in layers:
        x = act(jnp.dot(x, W, preferred_element_type=jnp.float32) + b)
    return x


def _flatten_params(params):
    names = ['pre_e_che', 'pre_e_vdw', 'pre_v', 'pre_u',
             'phi_e_che', 'phi_e_vdw', 'phi_v_che', 'phi_v_vdw',
             'phi_u_che', 'phi_u_vdw']
    arrs, counts = [], {}
    for nm in names:
        layers = params[nm]
        counts[nm] = len(layers)
        for (W, b) in layers:
            arrs.append(W)
            arrs.append(b.reshape(1, -1))
    return names, arrs, counts


def kernel(nodes, state, che_edges, vdw_edges, che_index, vdw_index,
           che_edge_index, vdw_edge_index, node_index, num_atoms,
           che_num_pairs, vdw_num_pairs, params):
    N, H = nodes.shape
    B = state.shape[0]
    APG = N // B
    E_CHE = che_edges.shape[0]
    E_VDW = vdw_edges.shape[0]
    ECPG = E_CHE // B
    EVPG = E_VDW // B

    G = _G if B % _G == 0 else 1
    GA = G * APG
    GEC = G * ECPG
    GEV = G * EVPG

    names, warrs, counts = _flatten_params(params)

    state_b = state.reshape(B, 1, H)
    chei_t = che_index.T.reshape(2, B // G, GEC).transpose(1, 0, 2)
    vdwi_t = vdw_index.T.reshape(2, B // G, GEV).transpose(1, 0, 2)
    cc = GEC // 8
    cv = GEV // 8
    che_i0 = che_index[:, 0].reshape(E_CHE // cc, cc)
    che_i1 = che_index[:, 1].reshape(E_CHE // cc, cc)
    vdw_i0 = vdw_index[:, 0].reshape(E_VDW // cv, cv)
    vdw_i1 = vdw_index[:, 1].reshape(E_VDW // cv, cv)

    def body(*refs):
        (nodes_ref, state_ref, che_ref, vdw_ref, chei_ref, vdwi_ref), rest = \
            refs[:6], refs[6:]
        nw = len(warrs)
        wrefs, outs = rest[:nw], rest[nw:]
        eche_o, evdw_o, v_o, u_o = outs

        layers = {}
        k = 0
        for nm in names:
            ls = []
            for _ in range(counts[nm]):
                ls.append((wrefs[k][...], wrefs[k + 1][...]))
                k += 2
            layers[nm] = ls

        pid = pl.program_id(0)
        base = pid * GA

        v_in = nodes_ref[...]
        u_in = state_ref[...].reshape(G, H)
        eche_in = che_ref[...]
        evdw_in = vdw_ref[...]

        v = _mlp(layers['pre_v'], v_in, _ssp_fast)
        u = _mlp(layers['pre_u'], u_in, _ssp_fast)
        e_che = _mlp(layers['pre_e_che'], eche_in, _ssp_fast)
        e_vdw = _mlp(layers['pre_e_vdw'], evdw_in, _ssp_fast)

        u_pernode = jnp.broadcast_to(u[:, None, :], (G, APG, H)).reshape(GA, H)

        def half(e, idx_ref, EPG, GE, phi_e, phi_v, phi_u):
            idxt = idx_ref[...].reshape(2, GE) - base
            iota_t = lax.broadcasted_iota(jnp.int32, (GA, GE), 0)
            ohg0t = (iota_t == idxt[0:1, :]).astype(jnp.float32)
            ohg1t = (iota_t == idxt[1:2, :]).astype(jnp.float32)
            tdot = (((0,), (0,)), ((), ()))
            center = lax.dot_general(ohg0t, v, tdot,
                                     preferred_element_type=jnp.float32)
            nbr = lax.dot_general(ohg1t, v, tdot,
                                  preferred_element_type=jnp.float32)
            u_e = jnp.broadcast_to(u[:, None, :], (G, EPG, H)).reshape(GE, H)
            e_p = _mlp(phi_e, jnp.concatenate([center, e, nbr, u_e], axis=-1), _ssp_fast)
            e_v = jnp.dot(ohg0t, e_p, preferred_element_type=jnp.float32)
            v_p = _mlp(phi_v, jnp.concatenate([e_v, v, u_pernode], axis=-1), _ssp_fast)
            e_u = jnp.sum(e_p.reshape(G, EPG, H), axis=1)
            v_u = jnp.sum(v_p.reshape(G, APG, H), axis=1)
            u_p = _mlp(phi_u, jnp.concatenate([e_u, v_u, u], axis=-1))
            return e_p, v_p, u_p

        che_e_p, che_v_p, che_u_p = half(
            e_che, chei_ref, ECPG, GEC,
            layers['phi_e_che'], layers['phi_v_che'], layers['phi_u_che'])
        vdw_e_p, vdw_v_p, vdw_u_p = half(
            e_vdw, vdwi_ref, EVPG, GEV,
            layers['phi_e_vdw'], layers['phi_v_vdw'], layers['phi_u_vdw'])

        eche_o[...] = eche_in + che_e_p
        evdw_o[...] = evdw_in + vdw_e_p
        v_o[...] = v_in + che_v_p + vdw_v_p
        u_o[...] = (u_in + che_u_p + vdw_u_p).reshape(G, 1, H)

    grid = (B // G,)

    def blk(shape):
        nd = len(shape)
        return pl.BlockSpec((G,) + shape[1:], lambda i: (i,) + (0,) * (nd - 1))

    def full(shape):
        nd = len(shape)
        return pl.BlockSpec(shape, lambda i: (0,) * nd)

    def row_blk(rows, cols):
        return pl.BlockSpec((rows, cols), lambda i: (i, 0))

    in_specs = [
        row_blk(GA, H), blk(state_b.shape), row_blk(GEC, H),
        row_blk(GEV, H),
        pl.BlockSpec((1, 2, GEC), lambda i: (i, 0, 0)),
        pl.BlockSpec((1, 2, GEV), lambda i: (i, 0, 0)),
    ] + [full(w.shape) for w in warrs]

    out_shape = (
        jax.ShapeDtypeStruct((E_CHE, H), jnp.float32),
        jax.ShapeDtypeStruct((E_VDW, H), jnp.float32),
        jax.ShapeDtypeStruct((N, H), jnp.float32),
        jax.ShapeDtypeStruct((B, 1, H), jnp.float32),
    )
    out_specs = (
        row_blk(GEC, H), row_blk(GEV, H), row_blk(GA, H), blk((B, 1, H)),
    )

    eche_o, evdw_o, v_o, u_o = pl.pallas_call(
        body,
        grid=grid,
        in_specs=in_specs,
        out_specs=out_specs,
        out_shape=out_shape,
        compiler_params=pltpu.CompilerParams(
            dimension_semantics=("parallel",),
        ),
    )(nodes, state_b, che_edges, vdw_edges, chei_t, vdwi_t, *warrs)

    return (eche_o, evdw_o, v_o, u_o.reshape(B, H))
